# baseline (speedup 1.0000x reference)
import functools

import jax
import jax.numpy as jnp
from jax import lax
from jax.experimental import pallas as pl
from jax.experimental.pallas import tpu as pltpu
from jax.experimental.pallas import tpu_sc as plsc

NC = 2
NS = 16
NW = NC * NS
LN_EPS = 1e-5


def _make_deg_kernel(EPAD, NPAD, C1):
  mesh = plsc.VectorSubcoreMesh(core_axis_name="c", subcore_axis_name="s")
  TS = NPAD // NS
  ep = EPAD // NW

  @functools.partial(
      pl.kernel,
      out_type=jax.ShapeDtypeStruct((NC, NPAD), jnp.float32),
      mesh=mesh,
      compiler_params=pltpu.CompilerParams(needs_layout_passes=False),
      scratch_types=[
          pltpu.VMEM((C1,), jnp.int32),
          pltpu.VMEM((C1,), jnp.float32),
          pltpu.VMEM((NPAD,), jnp.float32),
          pltpu.VMEM_SHARED((NS, NPAD), jnp.float32),
          pltpu.VMEM((NS, TS), jnp.float32),
      ],
  )
  def deg_kernel(col_hbm, ew_hbm, out_hbm, colv, ewv, degl, stage, redbuf):
    cid = lax.axis_index("c")
    sid = lax.axis_index("s")
    wid = sid * NC + cid

    def z(i, _):
      degl[pl.ds(i * 16, 16)] = jnp.zeros((16,), jnp.float32)
      return 0
    lax.fori_loop(0, NPAD // 16, z, 0)

    base = wid * ep

    def chunk(i, _):
      off = base + i * C1
      pltpu.sync_copy(col_hbm.at[pl.ds(off, C1)], colv)
      pltpu.sync_copy(ew_hbm.at[pl.ds(off, C1)], ewv)

      def inner(j, _):
        idx = colv[pl.ds(j * 16, 16)]
        w = ewv[pl.ds(j * 16, 16)]
        plsc.addupdate_scatter(degl, [idx], w)
        return 0
      lax.fori_loop(0, C1 // 16, inner, 0)
      return 0
    lax.fori_loop(0, ep // C1, chunk, 0)

    pltpu.sync_copy(degl, stage.at[sid])
    plsc.subcore_barrier()
    for t in range(NS):
      pltpu.sync_copy(stage.at[t, pl.ds(sid * TS, TS)], redbuf.at[t])

    def red(j, _):
      s = redbuf[0, pl.ds(j * 16, 16)]
      for t in range(1, NS):
        s = s + redbuf[t, pl.ds(j * 16, 16)]
      degl[pl.ds(j * 16, 16)] = s
      return 0
    lax.fori_loop(0, TS // 16, red, 0)
    pltpu.sync_copy(degl.at[pl.ds(0, TS)], out_hbm.at[cid, pl.ds(sid * TS, TS)])

  return deg_kernel


def _make_edge_kernel(EPAD, NP, D, C, NCH0, NCH1):
  mesh = plsc.VectorSubcoreMesh(core_axis_name="c", subcore_axis_name="s")
  RPT = NP // NS
  ZR = C
  NZ = RPT // ZR
  NB = 4
  NPK = 4
  assert NCH0 + NCH1 == (EPAD // C) // NS
  assert NCH0 % NPK == 0 and NCH1 % NPK == 0
  assert min(NCH0, NCH1) >= 2 * NPK

  @functools.partial(
      pl.kernel,
      out_type=jax.ShapeDtypeStruct((NC, NP, D), jnp.float32),
      mesh=mesh,
      compiler_params=pltpu.CompilerParams(needs_layout_passes=False,
                                           use_tc_tiling_on_sc=False),
      scratch_types=[
          [pltpu.VMEM((4, C), jnp.int32) for _ in range(NPK)],
          [pltpu.VMEM((C, D // 2), jnp.uint32) for _ in range(NB)],
          [pltpu.VMEM((C, D), jnp.float32) for _ in range(2)],
          [pltpu.SemaphoreType.DMA for _ in range(NPK)],
          [pltpu.SemaphoreType.DMA for _ in range(NB)],
          [pltpu.SemaphoreType.DMA for _ in range(2)],
          pltpu.VMEM_SHARED((NP, D), jnp.float32),
      ],
  )
  def edge_kernel(pkt_hbm, g_hbm, out_hbm, pkt, bufs, msg, psem, gsem, ssem,
                  acc):
    cid = lax.axis_index("c")
    sid = lax.axis_index("s")
    NCHC = jnp.where(cid == 0, NCH0, NCH1)

    def zrow(i, _):
      for k in range(D // 16):
        msg[0][i, pl.ds(k * 16, 16)] = jnp.zeros((16,), jnp.float32)
      return 0
    lax.fori_loop(0, ZR, zrow, 0)
    for q in range(NZ):
      pltpu.sync_copy(msg[0], acc.at[pl.ds(sid * RPT + q * ZR, ZR)])
    plsc.subcore_barrier()

    cbase = jnp.where(cid == 0, sid * NCH0, NS * NCH0 + sid * NCH1)
    for p in range(3):
      pltpu.async_copy(pkt_hbm.at[cbase + p], pkt[p], psem[p])
    pltpu.make_async_copy(pkt_hbm.at[cbase], pkt[0], psem[0]).wait()
    pltpu.async_copy(g_hbm.at[pkt[0].at[0]], bufs[0], gsem[0])
    pltpu.make_async_copy(pkt_hbm.at[cbase + 1], pkt[1], psem[1]).wait()
    pltpu.async_copy(g_hbm.at[pkt[1].at[0]], bufs[1], gsem[1])

    def group(t, _):
      for b2 in range(NPK):
        i = t * NPK + b2

        @pl.when(i >= 1)
        def _():
          pltpu.make_async_copy(
              msg[(b2 - 1) % 2], acc.at[pkt[(b2 - 1) % NPK].at[1]],
              ssem[(b2 - 1) % 2]).wait()

        @pl.when(i + 3 < NCHC)
        def _():
          pltpu.async_copy(pkt_hbm.at[cbase + i + 3],
                           pkt[(b2 + 3) % NPK], psem[(b2 + 3) % NPK])

        @pl.when(i + 2 < NCHC)
        def _():
          pltpu.make_async_copy(pkt_hbm.at[cbase + i + 2],
                                pkt[(b2 + 2) % NPK],
                                psem[(b2 + 2) % NPK]).wait()
          pltpu.async_copy(g_hbm.at[pkt[(b2 + 2) % NPK].at[0]],
                           bufs[(b2 + 2) % NB], gsem[(b2 + 2) % NB])

        pltpu.make_async_copy(g_hbm.at[pkt[b2].at[0]], bufs[b2],
                              gsem[b2]).wait()

        hmask = jnp.full((16,), 0xFFFF0000, jnp.uint32)

        def scale(j, _):
          wv = plsc.bitcast(pkt[b2][2, pl.ds(j * 16, 16)], jnp.float32)
          for l in range(16):
            s = wv[l]
            e = j * 16 + l
            for k in range(D // 32):
              w = bufs[b2][e, pl.ds(k * 16, 16)]
              flo = plsc.bitcast(w << 16, jnp.float32)
              fhi = plsc.bitcast(w & hmask, jnp.float32)
              msg[b2 % 2][e, pl.ds(k * 16, 16)] = flo * s
              msg[b2 % 2][e, pl.ds(D // 2 + k * 16, 16)] = fhi * s
          return 0
        lax.fori_loop(0, C // 16, scale, 0)

        pltpu.async_copy(msg[b2 % 2], acc.at[pkt[b2].at[1]], ssem[b2 % 2],
                         add=True)
      return 0
    lax.fori_loop(0, NCHC // NPK, group, 0)
    pltpu.make_async_copy(msg[1], acc.at[pkt[NPK - 1].at[1]],
                          ssem[1]).wait()

    plsc.subcore_barrier()
    pltpu.sync_copy(acc.at[pl.ds(sid * RPT, RPT)],
                    out_hbm.at[cid, pl.ds(sid * RPT, RPT)])

  return edge_kernel


def _tc_pre(x, w, degp_t, BLK):
  N, D = x.shape

  def body(x_ref, w_ref, dg_ref, inp_ref, h_ref, gp_ref):
    xb = x_ref[...]
    mu = jnp.mean(xb, axis=1, keepdims=True)
    xc = xb - mu
    var = jnp.mean(xc * xc, axis=1, keepdims=True)
    inp = xc * lax.rsqrt(var + LN_EPS)
    inp_ref[...] = inp
    h = jnp.dot(inp, w_ref[...], preferred_element_type=jnp.float32)
    h_ref[...] = h
    dg = dg_ref[...]
    deg = 1.0 + dg[:, 0:1] + dg[:, 1:2]
    dis = lax.rsqrt(deg)
    g = h * dis
    gb = g.astype(jnp.bfloat16)
    lo = lax.bitcast_convert_type(gb[:, :D // 2], jnp.uint16)
    hi = lax.bitcast_convert_type(gb[:, D // 2:], jnp.uint16)
    gp_ref[...] = lo.astype(jnp.uint32) | (hi.astype(jnp.uint32) << 16)

  return pl.pallas_call(
      body,
      grid=(N // BLK,),
      in_specs=[
          pl.BlockSpec((BLK, D), lambda i: (i, 0)),
          pl.BlockSpec((D, D), lambda i: (0, 0)),
          pl.BlockSpec((BLK, 2), lambda i: (i, 0)),
      ],
      out_specs=[
          pl.BlockSpec((BLK, D), lambda i: (i, 0)),
          pl.BlockSpec((BLK, D), lambda i: (i, 0)),
          pl.BlockSpec((BLK, D // 2), lambda i: (i, 0)),
      ],
      out_shape=[
          jax.ShapeDtypeStruct((N, D), jnp.float32),
          jax.ShapeDtypeStruct((N, D), jnp.float32),
          jax.ShapeDtypeStruct((N, D // 2), jnp.uint32),
      ],
  )(x, w, degp_t)


def _tc_post(parts, h, ln_x, degp_t, b2, BLK):
  N, D = h.shape

  def body(p_ref, h_ref, inp_ref, dg_ref, b_ref, o_ref):
    dg = dg_ref[...]
    deg = 1.0 + dg[:, 0:1] + dg[:, 1:2]
    dis = lax.rsqrt(deg)
    s = p_ref[0] + p_ref[1]
    o_ref[...] = (dis * s + (dis * dis) * h_ref[...] + b_ref[...]
                  - inp_ref[...])

  return pl.pallas_call(
      body,
      grid=(N // BLK,),
      in_specs=[
          pl.BlockSpec((2, BLK, D), lambda i: (0, i, 0)),
          pl.BlockSpec((BLK, D), lambda i: (i, 0)),
          pl.BlockSpec((BLK, D), lambda i: (i, 0)),
          pl.BlockSpec((BLK, 2), lambda i: (i, 0)),
          pl.BlockSpec((1, D), lambda i: (0, 0)),
      ],
      out_specs=pl.BlockSpec((BLK, D), lambda i: (i, 0)),
      out_shape=jax.ShapeDtypeStruct((N, D), jnp.float32),
  )(parts, h, ln_x, degp_t, b2)


def kernel(node_inputs, edge_index, edge_weights, W, b):
  N, D = node_inputs.shape
  E = edge_weights.shape[0]

  row = edge_index[0].astype(jnp.int32)
  col = edge_index[1].astype(jnp.int32)
  ew = edge_weights.astype(jnp.float32)

  C = 64
  C1 = 2048
  unit = NW * C1
  EPAD = ((E + unit - 1) // unit) * unit
  if EPAD != E:
    pad = EPAD - E
    row = jnp.concatenate([row, jnp.zeros((pad,), jnp.int32)])
    col = jnp.concatenate([col, jnp.zeros((pad,), jnp.int32)])
    ew = jnp.concatenate([ew, jnp.zeros((pad,), jnp.float32)])

  NPAD = ((N + 255) // 256) * 256

  degp = _make_deg_kernel(EPAD, NPAD, C1)(col, ew)
  degp_t = degp.T[:N].reshape(N, 2)

  BLK = 1000
  ln_x, h, gp = _tc_pre(node_inputs.astype(jnp.float32), W.astype(jnp.float32),
                        degp_t, BLK)

  ew_bits = lax.bitcast_convert_type(ew, jnp.int32)
  packed = jnp.stack(
      [row.reshape(EPAD // C, C), col.reshape(EPAD // C, C),
       ew_bits.reshape(EPAD // C, C),
       jnp.zeros((EPAD // C, C), jnp.int32)], axis=1)
  NCH0, NCH1 = 224, 96
  parts = _make_edge_kernel(EPAD, NPAD, D, C, NCH0, NCH1)(packed, gp)

  return _tc_post(parts, h, ln_x, degp_t, b.reshape(1, D).astype(jnp.float32),
                  BLK)


# speedup vs baseline: 1.0002x; 1.0002x over previous
import functools

import jax
import jax.numpy as jnp
from jax import lax
from jax.experimental import pallas as pl
from jax.experimental.pallas import tpu as pltpu
from jax.experimental.pallas import tpu_sc as plsc

NC = 2
NS = 16
NW = NC * NS
LN_EPS = 1e-5


def _make_deg_kernel(EPAD, NPAD, C1):
  mesh = plsc.VectorSubcoreMesh(core_axis_name="c", subcore_axis_name="s")
  TS = NPAD // NS
  ep = EPAD // NW

  @functools.partial(
      pl.kernel,
      out_type=jax.ShapeDtypeStruct((NC, NPAD), jnp.float32),
      mesh=mesh,
      compiler_params=pltpu.CompilerParams(needs_layout_passes=False),
      scratch_types=[
          pltpu.VMEM((C1,), jnp.int32),
          pltpu.VMEM((C1,), jnp.float32),
          pltpu.VMEM((NPAD,), jnp.float32),
          pltpu.VMEM_SHARED((NS, NPAD), jnp.float32),
          pltpu.VMEM((NS, TS), jnp.float32),
      ],
  )
  def deg_kernel(col_hbm, ew_hbm, out_hbm, colv, ewv, degl, stage, redbuf):
    cid = lax.axis_index("c")
    sid = lax.axis_index("s")
    wid = sid * NC + cid

    def z(i, _):
      degl[pl.ds(i * 16, 16)] = jnp.zeros((16,), jnp.float32)
      return 0
    lax.fori_loop(0, NPAD // 16, z, 0)

    base = wid * ep

    def chunk(i, _):
      off = base + i * C1
      pltpu.sync_copy(col_hbm.at[pl.ds(off, C1)], colv)
      pltpu.sync_copy(ew_hbm.at[pl.ds(off, C1)], ewv)

      def inner(j, _):
        idx = colv[pl.ds(j * 16, 16)]
        w = ewv[pl.ds(j * 16, 16)]
        plsc.addupdate_scatter(degl, [idx], w)
        return 0
      lax.fori_loop(0, C1 // 16, inner, 0)
      return 0
    lax.fori_loop(0, ep // C1, chunk, 0)

    pltpu.sync_copy(degl, stage.at[sid])
    plsc.subcore_barrier()
    for t in range(NS):
      pltpu.sync_copy(stage.at[t, pl.ds(sid * TS, TS)], redbuf.at[t])

    def red(j, _):
      s = redbuf[0, pl.ds(j * 16, 16)]
      for t in range(1, NS):
        s = s + redbuf[t, pl.ds(j * 16, 16)]
      degl[pl.ds(j * 16, 16)] = s
      return 0
    lax.fori_loop(0, TS // 16, red, 0)
    pltpu.sync_copy(degl.at[pl.ds(0, TS)], out_hbm.at[cid, pl.ds(sid * TS, TS)])

  return deg_kernel


def _make_edge_kernel(EPAD, NP, D, C, NCH0, NCH1):
  mesh = plsc.VectorSubcoreMesh(core_axis_name="c", subcore_axis_name="s")
  RPT = NP // NS
  ZR = C
  NZ = RPT // ZR
  NB = 4
  NPK = 4
  assert NCH0 + NCH1 == (EPAD // C) // NS
  assert NCH0 % NPK == 0 and NCH1 % NPK == 0
  assert min(NCH0, NCH1) >= 2 * NPK

  @functools.partial(
      pl.kernel,
      out_type=jax.ShapeDtypeStruct((NC, NP, D), jnp.float32),
      mesh=mesh,
      compiler_params=pltpu.CompilerParams(needs_layout_passes=False,
                                           use_tc_tiling_on_sc=False),
      scratch_types=[
          [pltpu.VMEM((4, C), jnp.int32) for _ in range(NPK)],
          [pltpu.VMEM((C, D // 2), jnp.uint32) for _ in range(NB)],
          [pltpu.VMEM((C, D), jnp.float32) for _ in range(2)],
          [pltpu.SemaphoreType.DMA for _ in range(NPK)],
          [pltpu.SemaphoreType.DMA for _ in range(NB)],
          [pltpu.SemaphoreType.DMA for _ in range(2)],
          pltpu.VMEM_SHARED((NP, D), jnp.float32),
      ],
  )
  def edge_kernel(pkt_hbm, g_hbm, out_hbm, pkt, bufs, msg, psem, gsem, ssem,
                  acc):
    cid = lax.axis_index("c")
    sid = lax.axis_index("s")
    NCHC = jnp.where(cid == 0, NCH0, NCH1)

    def zrow(i, _):
      for k in range(D // 16):
        msg[0][i, pl.ds(k * 16, 16)] = jnp.zeros((16,), jnp.float32)
      return 0
    lax.fori_loop(0, ZR, zrow, 0)
    for q in range(NZ):
      pltpu.sync_copy(msg[0], acc.at[pl.ds(sid * RPT + q * ZR, ZR)])
    plsc.subcore_barrier()

    cbase = jnp.where(cid == 0, sid * NCH0, NS * NCH0 + sid * NCH1)
    for p in range(3):
      pltpu.async_copy(pkt_hbm.at[cbase + p], pkt[p], psem[p])
    pltpu.make_async_copy(pkt_hbm.at[cbase], pkt[0], psem[0]).wait()
    pltpu.async_copy(g_hbm.at[pkt[0].at[0]], bufs[0], gsem[0])
    pltpu.make_async_copy(pkt_hbm.at[cbase + 1], pkt[1], psem[1]).wait()
    pltpu.async_copy(g_hbm.at[pkt[1].at[0]], bufs[1], gsem[1])

    def group(t, _):
      for b2 in range(NPK):
        i = t * NPK + b2

        @pl.when(i >= 1)
        def _():
          pltpu.make_async_copy(
              msg[(b2 - 1) % 2], acc.at[pkt[(b2 - 1) % NPK].at[1]],
              ssem[(b2 - 1) % 2]).wait()

        @pl.when(i + 3 < NCHC)
        def _():
          pltpu.async_copy(pkt_hbm.at[cbase + i + 3],
                           pkt[(b2 + 3) % NPK], psem[(b2 + 3) % NPK])

        @pl.when(i + 2 < NCHC)
        def _():
          pltpu.make_async_copy(pkt_hbm.at[cbase + i + 2],
                                pkt[(b2 + 2) % NPK],
                                psem[(b2 + 2) % NPK]).wait()
          pltpu.async_copy(g_hbm.at[pkt[(b2 + 2) % NPK].at[0]],
                           bufs[(b2 + 2) % NB], gsem[(b2 + 2) % NB])

        pltpu.make_async_copy(g_hbm.at[pkt[b2].at[0]], bufs[b2],
                              gsem[b2]).wait()

        hmask = jnp.full((16,), 0xFFFF0000, jnp.uint32)

        def scale(j, _):
          wv = plsc.bitcast(pkt[b2][2, pl.ds(j * 16, 16)], jnp.float32)
          for l in range(16):
            s = wv[l]
            e = j * 16 + l
            for k in range(D // 32):
              w = bufs[b2][e, pl.ds(k * 16, 16)]
              flo = plsc.bitcast(w << 16, jnp.float32)
              fhi = plsc.bitcast(w & hmask, jnp.float32)
              msg[b2 % 2][e, pl.ds(k * 16, 16)] = flo * s
              msg[b2 % 2][e, pl.ds(D // 2 + k * 16, 16)] = fhi * s
          return 0
        lax.fori_loop(0, C // 16, scale, 0)

        pltpu.async_copy(msg[b2 % 2], acc.at[pkt[b2].at[1]], ssem[b2 % 2],
                         add=True)
      return 0
    lax.fori_loop(0, NCHC // NPK, group, 0)
    pltpu.make_async_copy(msg[1], acc.at[pkt[NPK - 1].at[1]],
                          ssem[1]).wait()

    plsc.subcore_barrier()
    pltpu.sync_copy(acc.at[pl.ds(sid * RPT, RPT)],
                    out_hbm.at[cid, pl.ds(sid * RPT, RPT)])

  return edge_kernel


def _tc_pre(x, w, degp_t, BLK):
  N, D = x.shape

  def body(x_ref, w_ref, dg_ref, inp_ref, h_ref, gp_ref):
    xb = x_ref[...]
    mu = jnp.mean(xb, axis=1, keepdims=True)
    xc = xb - mu
    var = jnp.mean(xc * xc, axis=1, keepdims=True)
    inp = xc * lax.rsqrt(var + LN_EPS)
    inp_ref[...] = inp
    h = jnp.dot(inp, w_ref[...], preferred_element_type=jnp.float32)
    h_ref[...] = h
    dg = dg_ref[...]
    deg = 1.0 + dg[:, 0:1] + dg[:, 1:2]
    dis = lax.rsqrt(deg)
    g = h * dis
    gb = g.astype(jnp.bfloat16)
    lo = lax.bitcast_convert_type(gb[:, :D // 2], jnp.uint16)
    hi = lax.bitcast_convert_type(gb[:, D // 2:], jnp.uint16)
    gp_ref[...] = lo.astype(jnp.uint32) | (hi.astype(jnp.uint32) << 16)

  return pl.pallas_call(
      body,
      grid=(N // BLK,),
      in_specs=[
          pl.BlockSpec((BLK, D), lambda i: (i, 0)),
          pl.BlockSpec((D, D), lambda i: (0, 0)),
          pl.BlockSpec((BLK, 2), lambda i: (i, 0)),
      ],
      out_specs=[
          pl.BlockSpec((BLK, D), lambda i: (i, 0)),
          pl.BlockSpec((BLK, D), lambda i: (i, 0)),
          pl.BlockSpec((BLK, D // 2), lambda i: (i, 0)),
      ],
      out_shape=[
          jax.ShapeDtypeStruct((N, D), jnp.float32),
          jax.ShapeDtypeStruct((N, D), jnp.float32),
          jax.ShapeDtypeStruct((N, D // 2), jnp.uint32),
      ],
  )(x, w, degp_t)


def _tc_post(parts, h, ln_x, degp_t, b2, BLK):
  N, D = h.shape

  def body(p_ref, h_ref, inp_ref, dg_ref, b_ref, o_ref):
    dg = dg_ref[...]
    deg = 1.0 + dg[:, 0:1] + dg[:, 1:2]
    dis = lax.rsqrt(deg)
    s = p_ref[0] + p_ref[1]
    o_ref[...] = (dis * s + (dis * dis) * h_ref[...] + b_ref[...]
                  - inp_ref[...])

  return pl.pallas_call(
      body,
      grid=(N // BLK,),
      in_specs=[
          pl.BlockSpec((2, BLK, D), lambda i: (0, i, 0)),
          pl.BlockSpec((BLK, D), lambda i: (i, 0)),
          pl.BlockSpec((BLK, D), lambda i: (i, 0)),
          pl.BlockSpec((BLK, 2), lambda i: (i, 0)),
          pl.BlockSpec((1, D), lambda i: (0, 0)),
      ],
      out_specs=pl.BlockSpec((BLK, D), lambda i: (i, 0)),
      out_shape=jax.ShapeDtypeStruct((N, D), jnp.float32),
  )(parts, h, ln_x, degp_t, b2)


def kernel(node_inputs, edge_index, edge_weights, W, b):
  N, D = node_inputs.shape
  E = edge_weights.shape[0]

  row = edge_index[0].astype(jnp.int32)
  col = edge_index[1].astype(jnp.int32)
  ew = edge_weights.astype(jnp.float32)

  C = 64
  C1 = 2048
  unit = NW * C1
  EPAD = ((E + unit - 1) // unit) * unit
  if EPAD != E:
    pad = EPAD - E
    row = jnp.concatenate([row, jnp.zeros((pad,), jnp.int32)])
    col = jnp.concatenate([col, jnp.zeros((pad,), jnp.int32)])
    ew = jnp.concatenate([ew, jnp.zeros((pad,), jnp.float32)])

  NPAD = ((N + 255) // 256) * 256

  degp = _make_deg_kernel(EPAD, NPAD, C1)(col, ew)
  degp_t = degp.T[:N].reshape(N, 2)

  BLK = 1000
  ln_x, h, gp = _tc_pre(node_inputs.astype(jnp.float32), W.astype(jnp.float32),
                        degp_t, BLK)

  ew_bits = lax.bitcast_convert_type(ew, jnp.int32)
  packed = jnp.stack(
      [row.reshape(EPAD // C, C), col.reshape(EPAD // C, C),
       ew_bits.reshape(EPAD // C, C),
       jnp.zeros((EPAD // C, C), jnp.int32)], axis=1)
  NCH0, NCH1 = 96, 224
  parts = _make_edge_kernel(EPAD, NPAD, D, C, NCH0, NCH1)(packed, gp)

  return _tc_post(parts, h, ln_x, degp_t, b.reshape(1, D).astype(jnp.float32),
                  BLK)


# speedup vs baseline: 1.2867x; 1.2864x over previous
import functools

import jax
import jax.numpy as jnp
from jax import lax
from jax.experimental import pallas as pl
from jax.experimental.pallas import tpu as pltpu
from jax.experimental.pallas import tpu_sc as plsc

NC = 2
NS = 16
NW = NC * NS
LN_EPS = 1e-5


def _make_deg_kernel(EPAD, NPAD, C1):
  mesh = plsc.VectorSubcoreMesh(core_axis_name="c", subcore_axis_name="s")
  TS = NPAD // NS
  ep = EPAD // NW

  @functools.partial(
      pl.kernel,
      out_type=jax.ShapeDtypeStruct((NC, NPAD), jnp.float32),
      mesh=mesh,
      compiler_params=pltpu.CompilerParams(needs_layout_passes=False),
      scratch_types=[
          pltpu.VMEM((C1,), jnp.int32),
          pltpu.VMEM((C1,), jnp.float32),
          pltpu.VMEM((NPAD,), jnp.float32),
          pltpu.VMEM_SHARED((NS, NPAD), jnp.float32),
          pltpu.VMEM((NS, TS), jnp.float32),
      ],
  )
  def deg_kernel(col_hbm, ew_hbm, out_hbm, colv, ewv, degl, stage, redbuf):
    cid = lax.axis_index("c")
    sid = lax.axis_index("s")
    wid = sid * NC + cid

    def z(i, _):
      degl[pl.ds(i * 16, 16)] = jnp.zeros((16,), jnp.float32)
      return 0
    lax.fori_loop(0, NPAD // 16, z, 0)

    base = wid * ep

    def chunk(i, _):
      off = base + i * C1
      pltpu.sync_copy(col_hbm.at[pl.ds(off, C1)], colv)
      pltpu.sync_copy(ew_hbm.at[pl.ds(off, C1)], ewv)

      def inner(j, _):
        idx = colv[pl.ds(j * 16, 16)]
        w = ewv[pl.ds(j * 16, 16)]
        plsc.addupdate_scatter(degl, [idx], w)
        return 0
      lax.fori_loop(0, C1 // 16, inner, 0)
      return 0
    lax.fori_loop(0, ep // C1, chunk, 0)

    pltpu.sync_copy(degl, stage.at[sid])
    plsc.subcore_barrier()
    for t in range(NS):
      pltpu.sync_copy(stage.at[t, pl.ds(sid * TS, TS)], redbuf.at[t])

    def red(j, _):
      s = redbuf[0, pl.ds(j * 16, 16)]
      for t in range(1, NS):
        s = s + redbuf[t, pl.ds(j * 16, 16)]
      degl[pl.ds(j * 16, 16)] = s
      return 0
    lax.fori_loop(0, TS // 16, red, 0)
    pltpu.sync_copy(degl.at[pl.ds(0, TS)], out_hbm.at[cid, pl.ds(sid * TS, TS)])

  return deg_kernel


def _make_edge_kernel(EPAD, NP, D, C, NCH0, NCH1):
  mesh = plsc.VectorSubcoreMesh(core_axis_name="c", subcore_axis_name="s")
  RPT = NP // NS
  ZR = C
  NZ = RPT // ZR
  NB = 4
  NPK = 4
  assert NCH0 + NCH1 == (EPAD // C) // NS
  assert NCH0 % NPK == 0 and NCH1 % NPK == 0
  assert min(NCH0, NCH1) >= 2 * NPK

  @functools.partial(
      pl.kernel,
      out_type=jax.ShapeDtypeStruct((NC, NP, D), jnp.float32),
      mesh=mesh,
      compiler_params=pltpu.CompilerParams(needs_layout_passes=False,
                                           use_tc_tiling_on_sc=False),
      scratch_types=[
          [pltpu.VMEM((4, C), jnp.int32) for _ in range(NPK)],
          [pltpu.VMEM((C, D // 2), jnp.uint32) for _ in range(NB)],
          [pltpu.VMEM((C, D), jnp.float32) for _ in range(2)],
          [pltpu.SemaphoreType.DMA for _ in range(NPK)],
          [pltpu.SemaphoreType.DMA for _ in range(NB)],
          [pltpu.SemaphoreType.DMA for _ in range(2)],
          pltpu.VMEM_SHARED((NP, D), jnp.float32),
      ],
  )
  def edge_kernel(pkt_hbm, g_hbm, out_hbm, pkt, bufs, msg, psem, gsem, ssem,
                  acc):
    cid = lax.axis_index("c")
    sid = lax.axis_index("s")
    NCHC = jnp.where(cid == 0, NCH0, NCH1)

    def zrow(i, _):
      for k in range(D // 16):
        msg[0][i, pl.ds(k * 16, 16)] = jnp.zeros((16,), jnp.float32)
      return 0
    lax.fori_loop(0, ZR, zrow, 0)
    for q in range(NZ):
      pltpu.sync_copy(msg[0], acc.at[pl.ds(sid * RPT + q * ZR, ZR)])
    plsc.subcore_barrier()

    cbase = jnp.where(cid == 0, sid * NCH0, NS * NCH0 + sid * NCH1)
    for p in range(3):
      pltpu.async_copy(pkt_hbm.at[cbase + p], pkt[p], psem[p])
    pltpu.make_async_copy(pkt_hbm.at[cbase], pkt[0], psem[0]).wait()
    pltpu.async_copy(g_hbm.at[pkt[0].at[0]], bufs[0], gsem[0])
    pltpu.make_async_copy(pkt_hbm.at[cbase + 1], pkt[1], psem[1]).wait()
    pltpu.async_copy(g_hbm.at[pkt[1].at[0]], bufs[1], gsem[1])

    def group(t, _):
      for b2 in range(NPK):
        i = t * NPK + b2

        @pl.when(i >= 1)
        def _():
          pltpu.make_async_copy(
              msg[(b2 - 1) % 2], acc.at[pkt[(b2 - 1) % NPK].at[1]],
              ssem[(b2 - 1) % 2]).wait()

        @pl.when(i + 3 < NCHC)
        def _():
          pltpu.async_copy(pkt_hbm.at[cbase + i + 3],
                           pkt[(b2 + 3) % NPK], psem[(b2 + 3) % NPK])

        @pl.when(i + 2 < NCHC)
        def _():
          pltpu.make_async_copy(pkt_hbm.at[cbase + i + 2],
                                pkt[(b2 + 2) % NPK],
                                psem[(b2 + 2) % NPK]).wait()
          pltpu.async_copy(g_hbm.at[pkt[(b2 + 2) % NPK].at[0]],
                           bufs[(b2 + 2) % NB], gsem[(b2 + 2) % NB])

        pltpu.make_async_copy(g_hbm.at[pkt[b2].at[0]], bufs[b2],
                              gsem[b2]).wait()

        hmask = jnp.full((16,), 0xFFFF0000, jnp.uint32)

        def scale(j, _):
          wv = plsc.bitcast(pkt[b2][2, pl.ds(j * 16, 16)], jnp.float32)
          for l in range(16):
            s = wv[l]
            e = j * 16 + l
            for k in range(D // 32):
              w = bufs[b2][e, pl.ds(k * 16, 16)]
              flo = plsc.bitcast(w << 16, jnp.float32)
              fhi = plsc.bitcast(w & hmask, jnp.float32)
              msg[b2 % 2][e, pl.ds(k * 16, 16)] = flo * s
              msg[b2 % 2][e, pl.ds(D // 2 + k * 16, 16)] = fhi * s
          return 0
        lax.fori_loop(0, C // 16, scale, 0)

        pltpu.async_copy(msg[b2 % 2], acc.at[pkt[b2].at[1]], ssem[b2 % 2],
                         add=True)
      return 0
    lax.fori_loop(0, NCHC // NPK, group, 0)
    pltpu.make_async_copy(msg[1], acc.at[pkt[NPK - 1].at[1]],
                          ssem[1]).wait()

    plsc.subcore_barrier()
    pltpu.sync_copy(acc.at[pl.ds(sid * RPT, RPT)],
                    out_hbm.at[cid, pl.ds(sid * RPT, RPT)])

  return edge_kernel


def _tc_pre(x, w, degp_t, BLK):
  N, D = x.shape

  def body(x_ref, w_ref, dg_ref, inp_ref, h_ref, gp_ref):
    xb = x_ref[...]
    mu = jnp.mean(xb, axis=1, keepdims=True)
    xc = xb - mu
    var = jnp.mean(xc * xc, axis=1, keepdims=True)
    inp = xc * lax.rsqrt(var + LN_EPS)
    inp_ref[...] = inp
    h = jnp.dot(inp, w_ref[...], preferred_element_type=jnp.float32)
    h_ref[...] = h
    dg = dg_ref[...]
    deg = 1.0 + dg[:, 0:1] + dg[:, 1:2]
    dis = lax.rsqrt(deg)
    g = h * dis
    gb = g.astype(jnp.bfloat16)
    lo = lax.bitcast_convert_type(gb[:, :D // 2], jnp.uint16)
    hi = lax.bitcast_convert_type(gb[:, D // 2:], jnp.uint16)
    gp_ref[...] = lo.astype(jnp.uint32) | (hi.astype(jnp.uint32) << 16)

  return pl.pallas_call(
      body,
      grid=(N // BLK,),
      in_specs=[
          pl.BlockSpec((BLK, D), lambda i: (i, 0)),
          pl.BlockSpec((D, D), lambda i: (0, 0)),
          pl.BlockSpec((BLK, 2), lambda i: (i, 0)),
      ],
      out_specs=[
          pl.BlockSpec((BLK, D), lambda i: (i, 0)),
          pl.BlockSpec((BLK, D), lambda i: (i, 0)),
          pl.BlockSpec((BLK, D // 2), lambda i: (i, 0)),
      ],
      out_shape=[
          jax.ShapeDtypeStruct((N, D), jnp.float32),
          jax.ShapeDtypeStruct((N, D), jnp.float32),
          jax.ShapeDtypeStruct((N, D // 2), jnp.uint32),
      ],
  )(x, w, degp_t)


def _tc_post(parts, h, ln_x, degp_t, b2, BLK):
  N, D = h.shape

  def body(p_ref, h_ref, inp_ref, dg_ref, b_ref, o_ref):
    dg = dg_ref[...]
    deg = 1.0 + dg[:, 0:1] + dg[:, 1:2]
    dis = lax.rsqrt(deg)
    s = p_ref[0] + p_ref[1]
    o_ref[...] = (dis * s + (dis * dis) * h_ref[...] + b_ref[...]
                  - inp_ref[...])

  return pl.pallas_call(
      body,
      grid=(N // BLK,),
      in_specs=[
          pl.BlockSpec((2, BLK, D), lambda i: (0, i, 0)),
          pl.BlockSpec((BLK, D), lambda i: (i, 0)),
          pl.BlockSpec((BLK, D), lambda i: (i, 0)),
          pl.BlockSpec((BLK, 2), lambda i: (i, 0)),
          pl.BlockSpec((1, D), lambda i: (0, 0)),
      ],
      out_specs=pl.BlockSpec((BLK, D), lambda i: (i, 0)),
      out_shape=jax.ShapeDtypeStruct((N, D), jnp.float32),
  )(parts, h, ln_x, degp_t, b2)


def kernel(node_inputs, edge_index, edge_weights, W, b):
  N, D = node_inputs.shape
  E = edge_weights.shape[0]

  row = edge_index[0].astype(jnp.int32)
  col = edge_index[1].astype(jnp.int32)
  ew = edge_weights.astype(jnp.float32)

  C = 64
  C1 = 2048
  unit = NW * C1
  EPAD = ((E + unit - 1) // unit) * unit
  if EPAD != E:
    pad = EPAD - E
    row = jnp.concatenate([row, jnp.zeros((pad,), jnp.int32)])
    col = jnp.concatenate([col, jnp.zeros((pad,), jnp.int32)])
    ew = jnp.concatenate([ew, jnp.zeros((pad,), jnp.float32)])

  NPAD = ((N + 255) // 256) * 256

  degp = _make_deg_kernel(EPAD, NPAD, C1)(col, ew)
  degp_t = degp.T[:N].reshape(N, 2)

  BLK = 1000
  ln_x, h, gp = _tc_pre(node_inputs.astype(jnp.float32), W.astype(jnp.float32),
                        degp_t, BLK)

  ew_bits = lax.bitcast_convert_type(ew, jnp.int32)
  packed = jnp.stack(
      [row.reshape(EPAD // C, C), col.reshape(EPAD // C, C),
       ew_bits.reshape(EPAD // C, C),
       jnp.zeros((EPAD // C, C), jnp.int32)], axis=1)
  NCH0, NCH1 = 160, 160
  parts = _make_edge_kernel(EPAD, NPAD, D, C, NCH0, NCH1)(packed, gp)

  return _tc_post(parts, h, ln_x, degp_t, b.reshape(1, D).astype(jnp.float32),
                  BLK)
